# baseline (speedup 1.0000x reference)
import functools

import jax
import jax.numpy as jnp
from jax import lax
from jax.experimental import pallas as pl
from jax.experimental.pallas import tpu as pltpu
from jax.experimental.pallas import tpu_sc as plsc

BATCH = 16384
EMBED_DIM = 64
H1 = 128
H2 = 64


@functools.cache
def _sc_info():
    info = plsc.get_sparse_core_info()
    return info.num_cores, info.num_subcores


@functools.cache
def _build_gather_linear():
    nc, ns = _sc_info()
    bpw = BATCH // (nc * ns)
    mesh = plsc.VectorSubcoreMesh(core_axis_name="c", subcore_axis_name="s")

    @functools.partial(
        pl.kernel,
        mesh=mesh,
        compiler_params=pltpu.CompilerParams(use_tc_tiling_on_sc=False),
        out_type=jax.ShapeDtypeStruct((BATCH, EMBED_DIM), jnp.float32),
        scratch_types=[
            pltpu.VMEM((bpw,), jnp.int32),
            pltpu.VMEM((bpw, EMBED_DIM), jnp.float32),
            pltpu.SemaphoreType.DMA,
        ],
    )
    def gather_lin(tab, idx, out, idx_v, rows_v, sem):
        wid = lax.axis_index("s") * nc + lax.axis_index("c")
        base = wid * bpw
        pltpu.sync_copy(idx.at[pl.ds(base, bpw)], idx_v)
        pltpu.async_copy(tab.at[idx_v], rows_v, sem).wait()
        pltpu.sync_copy(rows_v, out.at[pl.ds(base, bpw)])

    return gather_lin


@functools.cache
def _build_gather_tiled():
    nc, ns = _sc_info()
    bpw = BATCH // (nc * ns)
    mesh = plsc.VectorSubcoreMesh(core_axis_name="c", subcore_axis_name="s")

    @functools.partial(
        pl.kernel,
        mesh=mesh,
        out_type=jax.ShapeDtypeStruct((BATCH, EMBED_DIM), jnp.float32),
        scratch_types=[
            pltpu.VMEM((bpw,), jnp.int32),
            pltpu.VMEM((bpw, EMBED_DIM), jnp.float32),
            pltpu.SemaphoreType.DMA,
        ],
    )
    def gather_rows(tab, idx, out, idx_v, rows_v, sem):
        wid = lax.axis_index("s") * nc + lax.axis_index("c")
        base = wid * bpw
        pltpu.sync_copy(idx.at[pl.ds(base, bpw)], idx_v)

        def body(b, _):
            k = b * 16
            v = idx_v[pl.ds(k, 16)]
            for j in range(16):
                pltpu.async_copy(
                    tab.at[pl.ds(v[j], 1)], rows_v.at[pl.ds(k + j, 1)], sem)
            return 0

        lax.fori_loop(0, bpw // 16, body, 0)
        pltpu.make_async_copy(tab.at[pl.ds(0, bpw)], rows_v, sem).wait()
        pltpu.sync_copy(rows_v, out.at[pl.ds(base, bpw)])

    return gather_rows


def _mlp_body(ue_ref, se_ref, w1u_ref, w1s_ref, b1_ref, w2_ref, b2_ref,
              w3_ref, b3_ref, out_ref):
    dn1 = (((1,), (1,)), ((), ()))
    dn0 = (((1,), (0,)), ((), ()))
    h = lax.dot_general(w1u_ref[...], ue_ref[...], dn1,
                        preferred_element_type=jnp.float32)
    h += lax.dot_general(w1s_ref[...], se_ref[...], dn1,
                         preferred_element_type=jnp.float32)
    h = jnp.maximum(h + b1_ref[...], 0.0)
    h = lax.dot_general(w2_ref[...], h, dn0,
                        preferred_element_type=jnp.float32)
    h = jnp.maximum(h + b2_ref[...], 0.0)
    o = lax.dot_general(w3_ref[...], h, dn0,
                        preferred_element_type=jnp.float32)
    o = o + b3_ref[...]
    out_ref[...] = 1.0 / (1.0 + jnp.exp(-o))


def _mlp(ue, se, W1u, W1s, b1, W2, b2, W3, b3):
    blk = 2048
    grid = BATCH // blk
    full = lambda shape: pl.BlockSpec(shape, lambda i: (0, 0))
    return pl.pallas_call(
        _mlp_body,
        grid=(grid,),
        in_specs=[
            pl.BlockSpec((blk, EMBED_DIM), lambda i: (i, 0)),
            pl.BlockSpec((blk, EMBED_DIM), lambda i: (i, 0)),
            full((H1, EMBED_DIM)),
            full((H1, EMBED_DIM)),
            full((H1, 1)),
            full((H2, H1)),
            full((H2, 1)),
            full((1, H2)),
            full((1, 1)),
        ],
        out_specs=pl.BlockSpec((1, blk), lambda i: (0, i)),
        out_shape=jax.ShapeDtypeStruct((1, BATCH), jnp.float32),
    )(ue, se, W1u, W1s, b1, W2, b2, W3, b3)


def kernel(user, song, user_table, song_table, W1, b1, W2, b2, W3, b3):
    se = _build_gather_tiled()(song_table, song.astype(jnp.int32))
    ue = _build_gather_linear()(user_table, user.astype(jnp.int32))
    out = _mlp(ue, se, W1[:, :EMBED_DIM], W1[:, EMBED_DIM:], b1.reshape(H1, 1),
               W2, b2.reshape(H2, 1), W3, b3.reshape(1, 1))
    return out.reshape(BATCH, 1)


# speedup vs baseline: 1.3024x; 1.3024x over previous
import functools

import jax
import jax.numpy as jnp
from jax import lax
from jax.experimental import pallas as pl
from jax.experimental.pallas import tpu as pltpu
from jax.experimental.pallas import tpu_sc as plsc

BATCH = 16384
EMBED_DIM = 64
H1 = 128
H2 = 64


@functools.cache
def _sc_info():
    info = plsc.get_sparse_core_info()
    return info.num_cores, info.num_subcores


@functools.cache
def _build_gather_linear():
    nc, ns = _sc_info()
    bpw = BATCH // (nc * ns)
    mesh = plsc.VectorSubcoreMesh(core_axis_name="c", subcore_axis_name="s")

    @functools.partial(
        pl.kernel,
        mesh=mesh,
        compiler_params=pltpu.CompilerParams(use_tc_tiling_on_sc=False),
        out_type=jax.ShapeDtypeStruct((BATCH, EMBED_DIM), jnp.float32),
        scratch_types=[
            pltpu.VMEM((bpw,), jnp.int32),
            pltpu.VMEM((bpw, EMBED_DIM), jnp.float32),
            pltpu.SemaphoreType.DMA,
        ],
    )
    def gather_lin(tab, idx, out, idx_v, rows_v, sem):
        wid = lax.axis_index("s") * nc + lax.axis_index("c")
        base = wid * bpw
        pltpu.sync_copy(idx.at[pl.ds(base, bpw)], idx_v)
        pltpu.async_copy(tab.at[idx_v], rows_v, sem).wait()
        pltpu.sync_copy(rows_v, out.at[pl.ds(base, bpw)])

    return gather_lin


@functools.cache
def _build_gather_tiled():
    nc, ns = _sc_info()
    bpw = BATCH // (nc * ns)
    mesh = plsc.VectorSubcoreMesh(core_axis_name="c", subcore_axis_name="s")

    @functools.partial(
        pl.kernel,
        mesh=mesh,
        out_type=jax.ShapeDtypeStruct((BATCH, EMBED_DIM), jnp.float32),
        scratch_types=[
            pltpu.VMEM((bpw,), jnp.int32),
            pltpu.VMEM((bpw, EMBED_DIM), jnp.float32),
            pltpu.SemaphoreType.DMA,
        ],
    )
    def gather_rows(tab, idx, out, idx_v, rows_v, sem):
        wid = lax.axis_index("s") * nc + lax.axis_index("c")
        base = wid * bpw
        pltpu.sync_copy(idx.at[pl.ds(base, bpw)], idx_v)

        def body(b, _):
            k = b * 16
            v = idx_v[pl.ds(k, 16)]
            for j in range(16):
                pltpu.async_copy(
                    tab.at[pl.ds(v[j], 1)], rows_v.at[pl.ds(k + j, 1)], sem)
            return 0

        lax.fori_loop(0, bpw // 16, body, 0)
        pltpu.make_async_copy(tab.at[pl.ds(0, bpw)], rows_v, sem).wait()
        pltpu.sync_copy(rows_v, out.at[pl.ds(base, bpw)])

    return gather_rows


def _mlp_body(ue_ref, se_ref, w1u_ref, w1s_ref, b1_ref, w2_ref, b2_ref,
              w3_ref, b3_ref, out_ref):
    dn1 = (((1,), (1,)), ((), ()))
    dn0 = (((1,), (0,)), ((), ()))
    h = lax.dot_general(w1u_ref[...], ue_ref[...], dn1,
                        preferred_element_type=jnp.float32)
    h += lax.dot_general(w1s_ref[...], se_ref[...], dn1,
                         preferred_element_type=jnp.float32)
    h = jnp.maximum(h + b1_ref[...], 0.0)
    h = lax.dot_general(w2_ref[...], h, dn0,
                        preferred_element_type=jnp.float32)
    h = jnp.maximum(h + b2_ref[...], 0.0)
    o = lax.dot_general(w3_ref[...], h, dn0,
                        preferred_element_type=jnp.float32)
    o = o + b3_ref[...]
    out_ref[...] = 1.0 / (1.0 + jnp.exp(-o))


def _mlp(ue, se, W1u, W1s, b1, W2, b2, W3, b3):
    blk = 4096
    grid = BATCH // blk
    full = lambda shape: pl.BlockSpec(shape, lambda i: (0, 0))
    return pl.pallas_call(
        _mlp_body,
        grid=(grid,),
        in_specs=[
            pl.BlockSpec((blk, EMBED_DIM), lambda i: (i, 0)),
            pl.BlockSpec((blk, EMBED_DIM), lambda i: (i, 0)),
            full((H1, EMBED_DIM)),
            full((H1, EMBED_DIM)),
            full((H1, 1)),
            full((H2, H1)),
            full((H2, 1)),
            full((1, H2)),
            full((1, 1)),
        ],
        out_specs=pl.BlockSpec((1, blk), lambda i: (0, i)),
        out_shape=jax.ShapeDtypeStruct((1, BATCH), jnp.float32),
    )(ue, se, W1u, W1s, b1, W2, b2, W3, b3)


def kernel(user, song, user_table, song_table, W1, b1, W2, b2, W3, b3):
    ue = _build_gather_tiled()(user_table, user.astype(jnp.int32))
    se = _build_gather_tiled()(song_table, song.astype(jnp.int32))
    out = _mlp(ue, se, W1[:, :EMBED_DIM], W1[:, EMBED_DIM:], b1.reshape(H1, 1),
               W2, b2.reshape(H2, 1), W3, b3.reshape(1, 1))
    return out.reshape(BATCH, 1)
